# baseline (speedup 1.0000x reference)
import functools

import jax
import jax.numpy as jnp
from jax import lax
from jax.experimental import pallas as pl
from jax.experimental.pallas import tpu as pltpu
from jax.experimental.pallas import tpu_sc as plsc

VOCAB = 1_000_000
HIDDEN = 32
BATCH = 4096
HIST = 50
NC, NS, LANES = 2, 16, 16
NW = NC * NS
BPW = BATCH // NW
NGRP = BPW // LANES
TAIL = (VOCAB // 128) * 128


def _wid_base():
    wid = lax.axis_index("s") * NC + lax.axis_index("c")
    return wid * BPW


def _ctx_body(ctxT_hbm, inT_hbm, con_hbm, ctx_v, win0, win1, acc, sem0, sem1):
    base = _wid_base()
    pltpu.sync_copy(ctxT_hbm.at[:, pl.ds(base, BPW)], ctx_v)
    sems = (sem0, sem1)
    wins = (win0, win1)
    iota = lax.iota(jnp.int32, LANES)

    def issue_grp(h, c):
        sem, win = sems[c % 2], wins[c % 2]
        chunk = ctx_v[h, pl.ds(c * LANES, LANES)]
        for l in range(LANES):
            b16 = pl.multiple_of(
                lax.shift_left(lax.shift_right_logical(chunk[l], 4), 4), 16)
            pltpu.async_copy(inT_hbm.at[:, pl.ds(b16, LANES)],
                             win.at[l, :, pl.ds(0, LANES)], sem)

    def drain_grp(c):
        sem, win = sems[c % 2], wins[c % 2]
        for _ in range(LANES):
            pltpu.make_async_copy(inT_hbm.at[:, pl.ds(0, LANES)],
                                  win.at[0, :, pl.ds(0, LANES)], sem).wait()

    def accum_grp(h, c):
        win = wins[c % 2]
        chunk = ctx_v[h, pl.ds(c * LANES, LANES)]
        for l in range(LANES):
            off = lax.bitwise_and(chunk[l], 15)
            offs = jnp.full((LANES,), off, jnp.int32)
            ls = jnp.full((LANES,), l, jnp.int32)
            slot = c * LANES + l
            lo = plsc.load_gather(win, [ls, iota, offs])
            hi = plsc.load_gather(win, [ls, iota + LANES, offs])
            plsc.addupdate(acc.at[slot, pl.ds(0, LANES)], lo)
            plsc.addupdate(acc.at[slot, pl.ds(LANES, LANES)], hi)

    zeros = jnp.zeros((LANES,), jnp.float32)

    @pl.loop(0, BPW)
    def _zero(i):
        acc[i, pl.ds(0, LANES)] = zeros
        acc[i, pl.ds(LANES, LANES)] = zeros

    @pl.loop(0, HIST)
    def _h(h):
        issue_grp(h, 0)
        for c in range(NGRP):
            if c + 1 < NGRP:
                issue_grp(h, c + 1)
            drain_grp(c)
            accum_grp(h, c)

    pltpu.sync_copy(acc, con_hbm.at[pl.ds(base, BPW)])


def _ctx_gather(ctxT, in_embT):
    mesh = plsc.VectorSubcoreMesh(core_axis_name="c", subcore_axis_name="s",
                                  num_cores=NC, num_subcores=NS)
    f = pl.kernel(
        _ctx_body,
        out_type=jax.ShapeDtypeStruct((BATCH, HIDDEN), jnp.float32),
        mesh=mesh,
        compiler_params=pltpu.CompilerParams(use_tc_tiling_on_sc=False,
                                             needs_layout_passes=False),
        scratch_types=[
            pltpu.VMEM((HIST, BPW), jnp.int32),
            pltpu.VMEM((LANES, HIDDEN, 17), jnp.float32),
            pltpu.VMEM((LANES, HIDDEN, 17), jnp.float32),
            pltpu.VMEM((BPW, HIDDEN), jnp.float32),
            pltpu.SemaphoreType.DMA,
            pltpu.SemaphoreType.DMA,
        ],
    )
    return f(ctxT, in_embT)


def _tgt_body(t_hbm, outT_hbm, tgt_hbm,
              tidx_v, blk0, blk1, tgt_v, sem0, sem1):
    base = _wid_base()
    pltpu.sync_copy(t_hbm.at[pl.ds(base, BPW)], tidx_v)
    iota = lax.iota(jnp.int32, LANES)

    def block_base(vs):
        vc = lax.min(vs, TAIL - 1)
        return pl.multiple_of(
            lax.shift_left(lax.shift_right_logical(vc, 7), 7), 128)

    def fetch(vs, blk, sem):
        pltpu.async_copy(outT_hbm.at[:, pl.ds(block_base(vs), 128)], blk, sem)

    def wait(blk, sem):
        pltpu.make_async_copy(outT_hbm.at[:, pl.ds(0, 128)], blk, sem).wait()

    def select(i, vs, blk):
        off = lax.min(vs, TAIL - 1) - block_base(vs)
        lo = plsc.load_gather(blk, [iota, jnp.full((LANES,), off, jnp.int32)])
        hi = plsc.load_gather(blk, [iota + LANES,
                                    jnp.full((LANES,), off, jnp.int32)])
        tgt_v[i, pl.ds(0, LANES)] = lo
        tgt_v[i, pl.ds(LANES, LANES)] = hi

    @pl.loop(0, NGRP)
    def _g(c):
        chunk = tidx_v[pl.ds(c * LANES, LANES)]
        fetch(chunk[0], blk0, sem0)
        for l in range(LANES):
            if l + 1 < LANES:
                fetch(chunk[l + 1], (blk0, blk1)[(l + 1) % 2],
                      (sem0, sem1)[(l + 1) % 2])
            blk, sem = (blk0, blk1)[l % 2], (sem0, sem1)[l % 2]
            wait(blk, sem)
            select(c * LANES + l, chunk[l], blk)

    pltpu.sync_copy(tgt_v, tgt_hbm.at[pl.ds(base, BPW)])


def _tgt_gather(t_flat, out_embT):
    mesh = plsc.VectorSubcoreMesh(core_axis_name="c", subcore_axis_name="s",
                                  num_cores=NC, num_subcores=NS)
    f = pl.kernel(
        _tgt_body,
        out_type=jax.ShapeDtypeStruct((BATCH, HIDDEN), jnp.float32),
        mesh=mesh,
        compiler_params=pltpu.CompilerParams(needs_layout_passes=False),
        scratch_types=[
            pltpu.VMEM((BPW,), jnp.int32),
            pltpu.VMEM((HIDDEN, 128), jnp.float32),
            pltpu.VMEM((HIDDEN, 128), jnp.float32),
            pltpu.VMEM((BPW, HIDDEN), jnp.float32),
            pltpu.SemaphoreType.DMA,
            pltpu.SemaphoreType.DMA,
        ],
    )
    return f(t_flat, out_embT)


def _mm_body(a_ref, b_ref, o_ref):
    o_ref[...] = lax.dot_general(a_ref[...], b_ref[...],
                                 (((1,), (1,)), ((), ())),
                                 preferred_element_type=jnp.float32)


def _tc_matmul(con, tgt):
    blk = 1024
    return pl.pallas_call(
        _mm_body,
        grid=(BATCH // blk, BATCH // blk),
        in_specs=[pl.BlockSpec((blk, HIDDEN), lambda i, j: (i, 0)),
                  pl.BlockSpec((blk, HIDDEN), lambda i, j: (j, 0))],
        out_specs=pl.BlockSpec((blk, blk), lambda i, j: (i, j)),
        out_shape=jax.ShapeDtypeStruct((BATCH, BATCH), jnp.float32),
    )(con, tgt)


def kernel(contexts, t, in_emb, out_emb):
    ctxT = contexts.T
    t_flat = t.reshape(BATCH)
    in_embT = in_emb.T
    out_embT = out_emb.T
    con = _ctx_gather(ctxT, in_embT)
    tgt = _tgt_gather(t_flat, out_embT)
    tail_tab = out_emb[TAIL:]
    tail_rows = jnp.take(tail_tab, jnp.clip(t_flat - TAIL, 0, VOCAB - TAIL - 1),
                         axis=0)
    tgt = jnp.where((t_flat >= TAIL)[:, None], tail_rows, tgt)
    return _tc_matmul(con, tgt)


# speedup vs baseline: 3.1782x; 3.1782x over previous
import functools

import jax
import jax.numpy as jnp
from jax import lax
from jax.experimental import pallas as pl
from jax.experimental.pallas import tpu as pltpu
from jax.experimental.pallas import tpu_sc as plsc

VOCAB = 1_000_000
HIDDEN = 32
BATCH = 4096
HIST = 50
NC, NS, LANES = 2, 16, 16
NW = NC * NS
BPW = BATCH // NW
NGRP = BPW // LANES
TAIL = (VOCAB // 128) * 128
NBIG = VOCAB // 4
SLABW = 512
NSLAB = TAIL // SLABW
SPAD = SLABW + 1


def _wid():
    return lax.axis_index("s") * NC + lax.axis_index("c")


def _mesh():
    return plsc.VectorSubcoreMesh(core_axis_name="c", subcore_axis_name="s",
                                  num_cores=NC, num_subcores=NS)


def _t0_body(inT_hbm, tail_hbm, tbl4_hbm, sbuf, obuf, tbuf):
    w = _wid()
    iota = lax.iota(jnp.int32, LANES)
    row_lo = iota
    row_hi = iota + LANES

    npass = (NSLAB + NW - 1) // NW

    @pl.loop(0, npass)
    def _p(p):
        slab = w + p * NW

        @pl.when(slab < NSLAB)
        def _do():
            pltpu.sync_copy(inT_hbm.at[:, pl.ds(slab * SLABW, SLABW)],
                            sbuf.at[:, pl.ds(0, SLABW)])

            @pl.loop(0, SLABW // 4)
            def _r(r):
                for q in range(8):
                    k = q // 2
                    rows = row_lo if q % 2 == 0 else row_hi
                    col = jnp.full((LANES,), 4 * r + k, jnp.int32)
                    obuf[r, pl.ds(q * LANES, LANES)] = \
                        plsc.load_gather(sbuf, [rows, col])

            pltpu.sync_copy(obuf, tbl4_hbm.at[pl.ds(slab * (SLABW // 4),
                                                    SLABW // 4)])

    @pl.when(w == NW - 1)
    def _tail():
        pltpu.sync_copy(tail_hbm, tbuf)
        for r in range(16):
            for q in range(8):
                k = q // 2
                h0 = 0 if q % 2 == 0 else LANES
                obuf[r, pl.ds(q * LANES, LANES)] = \
                    tbuf[4 * r + k, pl.ds(h0, LANES)]
        pltpu.sync_copy(obuf.at[pl.ds(0, 16)],
                        tbl4_hbm.at[pl.ds(NBIG - 16, 16)])


def _repack(in_embT, tail_in):
    f = pl.kernel(
        _t0_body,
        out_type=jax.ShapeDtypeStruct((NBIG, 128), jnp.float32),
        mesh=_mesh(),
        compiler_params=pltpu.CompilerParams(needs_layout_passes=False),
        scratch_types=[
            pltpu.VMEM((HIDDEN, SPAD), jnp.float32),
            pltpu.VMEM((SLABW // 4, 128), jnp.float32),
            pltpu.VMEM((64, HIDDEN), jnp.float32),
        ],
    )
    return f(in_embT, tail_in)


def _ctx_body(ctxT_hbm, tbl4_hbm, con_hbm,
              ctx_v, big_v, rows0, rows1, acc, sem0, sem1):
    base = _wid() * BPW
    iota = lax.iota(jnp.int32, LANES)
    pltpu.sync_copy(ctxT_hbm.at[:, pl.ds(base, BPW)], ctx_v)

    @pl.loop(0, HIST)
    def _prep(h):
        for c in range(NGRP):
            sl = pl.ds(c * LANES, LANES)
            big_v[h, sl] = lax.shift_right_logical(ctx_v[h, sl], 2)

    zeros = jnp.zeros((LANES,), jnp.float32)

    @pl.loop(0, BPW)
    def _zero(i):
        acc[i, pl.ds(0, LANES)] = zeros
        acc[i, pl.ds(LANES, LANES)] = zeros

    def accum(h, rows):
        for c in range(NGRP):
            chunk = ctx_v[h, pl.ds(c * LANES, LANES)]
            for l in range(LANES):
                slot = c * LANES + l
                off = lax.shift_left(lax.bitwise_and(chunk[l], 3), 5)
                ids = jnp.full((LANES,), slot, jnp.int32)
                offs = iota + off
                lo = plsc.load_gather(rows, [ids, offs])
                hi = plsc.load_gather(rows, [ids, offs + LANES])
                plsc.addupdate(acc.at[slot, pl.ds(0, LANES)], lo)
                plsc.addupdate(acc.at[slot, pl.ds(LANES, LANES)], hi)

    pltpu.async_copy(tbl4_hbm.at[big_v.at[0]], rows0, sem0)
    pltpu.async_copy(tbl4_hbm.at[big_v.at[1]], rows1, sem1)

    @pl.loop(0, HIST, step=2)
    def _h(h):
        for b, (rows, sem) in enumerate(((rows0, sem0), (rows1, sem1))):
            hc = h + b
            pltpu.make_async_copy(tbl4_hbm.at[big_v.at[hc]], rows, sem).wait()
            accum(hc, rows)

            @pl.when(hc + 2 < HIST)
            def _next():
                pltpu.async_copy(tbl4_hbm.at[big_v.at[hc + 2]], rows, sem)

    pltpu.sync_copy(acc, con_hbm.at[pl.ds(base, BPW)])


def _ctx_gather(ctxT, tbl4):
    f = pl.kernel(
        _ctx_body,
        out_type=jax.ShapeDtypeStruct((BATCH, HIDDEN), jnp.float32),
        mesh=_mesh(),
        compiler_params=pltpu.CompilerParams(needs_layout_passes=False),
        scratch_types=[
            pltpu.VMEM((HIST, BPW), jnp.int32),
            pltpu.VMEM((HIST, BPW), jnp.int32),
            pltpu.VMEM((BPW, 128), jnp.float32),
            pltpu.VMEM((BPW, 128), jnp.float32),
            pltpu.VMEM((BPW, HIDDEN), jnp.float32),
            pltpu.SemaphoreType.DMA,
            pltpu.SemaphoreType.DMA,
        ],
    )
    return f(ctxT, tbl4)


def _tgt_body(t_hbm, outT_hbm, tgt_hbm,
              tidx_v, blk0, blk1, tgt_v, sem0, sem1):
    base = _wid() * BPW
    pltpu.sync_copy(t_hbm.at[pl.ds(base, BPW)], tidx_v)
    iota = lax.iota(jnp.int32, LANES)

    def block_base(vs):
        vc = lax.min(vs, TAIL - 1)
        return pl.multiple_of(
            lax.shift_left(lax.shift_right_logical(vc, 7), 7), 128)

    def fetch(vs, blk, sem):
        pltpu.async_copy(outT_hbm.at[:, pl.ds(block_base(vs), 128)], blk, sem)

    def wait(blk, sem):
        pltpu.make_async_copy(outT_hbm.at[:, pl.ds(0, 128)], blk, sem).wait()

    def select(i, vs, blk):
        off = lax.min(vs, TAIL - 1) - block_base(vs)
        lo = plsc.load_gather(blk, [iota, jnp.full((LANES,), off, jnp.int32)])
        hi = plsc.load_gather(blk, [iota + LANES,
                                    jnp.full((LANES,), off, jnp.int32)])
        tgt_v[i, pl.ds(0, LANES)] = lo
        tgt_v[i, pl.ds(LANES, LANES)] = hi

    @pl.loop(0, NGRP)
    def _g(c):
        chunk = tidx_v[pl.ds(c * LANES, LANES)]
        fetch(chunk[0], blk0, sem0)
        for l in range(LANES):
            if l + 1 < LANES:
                fetch(chunk[l + 1], (blk0, blk1)[(l + 1) % 2],
                      (sem0, sem1)[(l + 1) % 2])
            blk, sem = (blk0, blk1)[l % 2], (sem0, sem1)[l % 2]
            wait(blk, sem)
            select(c * LANES + l, chunk[l], blk)

    pltpu.sync_copy(tgt_v, tgt_hbm.at[pl.ds(base, BPW)])


def _tgt_gather(t_flat, out_embT):
    f = pl.kernel(
        _tgt_body,
        out_type=jax.ShapeDtypeStruct((BATCH, HIDDEN), jnp.float32),
        mesh=_mesh(),
        compiler_params=pltpu.CompilerParams(needs_layout_passes=False),
        scratch_types=[
            pltpu.VMEM((BPW,), jnp.int32),
            pltpu.VMEM((HIDDEN, 128), jnp.float32),
            pltpu.VMEM((HIDDEN, 128), jnp.float32),
            pltpu.VMEM((BPW, HIDDEN), jnp.float32),
            pltpu.SemaphoreType.DMA,
            pltpu.SemaphoreType.DMA,
        ],
    )
    return f(t_flat, out_embT)


def _mm_body(a_ref, b_ref, o_ref):
    o_ref[...] = lax.dot_general(a_ref[...], b_ref[...],
                                 (((1,), (1,)), ((), ())),
                                 preferred_element_type=jnp.float32)


def _tc_matmul(con, tgt):
    blk = 1024
    return pl.pallas_call(
        _mm_body,
        grid=(BATCH // blk, BATCH // blk),
        in_specs=[pl.BlockSpec((blk, HIDDEN), lambda i, j: (i, 0)),
                  pl.BlockSpec((blk, HIDDEN), lambda i, j: (j, 0))],
        out_specs=pl.BlockSpec((blk, blk), lambda i, j: (i, j)),
        out_shape=jax.ShapeDtypeStruct((BATCH, BATCH), jnp.float32),
    )(con, tgt)


def kernel(contexts, t, in_emb, out_emb):
    ctxT = contexts.T
    t_flat = t.reshape(BATCH)
    in_embT = in_emb.T
    out_embT = out_emb.T
    tail_in = in_emb[TAIL:]
    tail_tab = out_emb[TAIL:]

    tbl4 = _repack(in_embT, tail_in)
    con = _ctx_gather(ctxT, tbl4)
    tgt = _tgt_gather(t_flat, out_embT)
    tail_rows = jnp.take(tail_tab,
                         jnp.clip(t_flat - TAIL, 0, VOCAB - TAIL - 1), axis=0)
    tgt = jnp.where((t_flat >= TAIL)[:, None], tail_rows, tgt)
    return _tc_matmul(con, tgt)


# speedup vs baseline: 3.6000x; 1.1327x over previous
import functools

import jax
import jax.numpy as jnp
from jax import lax
from jax.experimental import pallas as pl
from jax.experimental.pallas import tpu as pltpu
from jax.experimental.pallas import tpu_sc as plsc

VOCAB = 1_000_000
HIDDEN = 32
BATCH = 4096
HIST = 50
NC, NS, LANES = 2, 16, 16
NW = NC * NS
BPW = BATCH // NW
NGRP = BPW // LANES
TAIL = (VOCAB // 128) * 128
NBIG = VOCAB // 4
SLABW = 512
NSLAB = TAIL // SLABW
SPAD = SLABW + 1


def _wid():
    return lax.axis_index("s") * NC + lax.axis_index("c")


def _mesh():
    return plsc.VectorSubcoreMesh(core_axis_name="c", subcore_axis_name="s",
                                  num_cores=NC, num_subcores=NS)


def _t0_body(inT_hbm, tail_hbm, tbl4_hbm, sbuf0, sbuf1, obuf0, obuf1, tbuf,
             si0, si1, so0, so1):
    w = _wid()
    iota = lax.iota(jnp.int32, LANES)
    row_lo = iota
    row_hi = iota + LANES
    sbufs, obufs = (sbuf0, sbuf1), (obuf0, obuf1)
    sis, sos = (si0, si1), (so0, so1)
    npass = (NSLAB + NW - 1) // NW

    def start_in(p, par):
        @pl.when(w + p * NW < NSLAB)
        def _():
            pltpu.async_copy(inT_hbm.at[:, pl.ds((w + p * NW) * SLABW, SLABW)],
                             sbufs[par].at[:, pl.ds(0, SLABW)], sis[par])

    def wait_in(par):
        pltpu.make_async_copy(inT_hbm.at[:, pl.ds(0, SLABW)],
                              sbufs[par].at[:, pl.ds(0, SLABW)],
                              sis[par]).wait()

    def wait_out(par):
        pltpu.make_async_copy(obufs[par], tbl4_hbm.at[pl.ds(0, SLABW // 4)],
                              sos[par]).wait()

    def compute(par):
        sbuf, obuf = sbufs[par], obufs[par]

        @pl.loop(0, SLABW // 4, unroll=4)
        def _r(r):
            for q in range(8):
                k = q // 2
                rows = row_lo if q % 2 == 0 else row_hi
                col = jnp.full((LANES,), 4 * r + k, jnp.int32)
                obuf[r, pl.ds(q * LANES, LANES)] = \
                    plsc.load_gather(sbuf, [rows, col])

    start_in(0, 0)

    @pl.loop(0, (npass + 1) // 2)
    def _p2(p2):
        for par in range(2):
            p = p2 * 2 + par
            slab = w + p * NW

            @pl.when(slab < NSLAB)
            def _do():
                wait_in(par)
                start_in(p + 1, 1 - par)

                @pl.when(p >= 2)
                def _w():
                    wait_out(par)
                compute(par)
                pltpu.async_copy(obufs[par],
                                 tbl4_hbm.at[pl.ds(slab * (SLABW // 4),
                                                   SLABW // 4)], sos[par])

    for par in range(2):
        last = npass - 2 + par

        @pl.when(w + last * NW < NSLAB)
        def _fin():
            wait_out(par if last % 2 == par else 1 - par)

    @pl.when(w == NW - 1)
    def _tail():
        pltpu.sync_copy(tail_hbm, tbuf)
        for r in range(16):
            for q in range(8):
                k = q // 2
                h0 = 0 if q % 2 == 0 else LANES
                obuf0[r, pl.ds(q * LANES, LANES)] = \
                    tbuf[4 * r + k, pl.ds(h0, LANES)]
        pltpu.sync_copy(obuf0.at[pl.ds(0, 16)],
                        tbl4_hbm.at[pl.ds(NBIG - 16, 16)])


def _repack(in_embT, tail_in):
    f = pl.kernel(
        _t0_body,
        out_type=jax.ShapeDtypeStruct((NBIG, 128), jnp.float32),
        mesh=_mesh(),
        compiler_params=pltpu.CompilerParams(needs_layout_passes=False),
        scratch_types=[
            pltpu.VMEM((HIDDEN, SPAD), jnp.float32),
            pltpu.VMEM((HIDDEN, SPAD), jnp.float32),
            pltpu.VMEM((SLABW // 4, 128), jnp.float32),
            pltpu.VMEM((SLABW // 4, 128), jnp.float32),
            pltpu.VMEM((64, HIDDEN), jnp.float32),
            pltpu.SemaphoreType.DMA,
            pltpu.SemaphoreType.DMA,
            pltpu.SemaphoreType.DMA,
            pltpu.SemaphoreType.DMA,
        ],
    )
    return f(in_embT, tail_in)


def _ctx_body(ctxT_hbm, tbl4_hbm, con_hbm,
              ctx_v, big_v, rows0, rows1, acc, sem0, sem1):
    base = _wid() * BPW
    iota = lax.iota(jnp.int32, LANES)
    pltpu.sync_copy(ctxT_hbm.at[:, pl.ds(base, BPW)], ctx_v)

    @pl.loop(0, HIST)
    def _prep(h):
        for c in range(NGRP):
            sl = pl.ds(c * LANES, LANES)
            big_v[h, sl] = lax.shift_right_logical(ctx_v[h, sl], 2)

    zeros = jnp.zeros((LANES,), jnp.float32)

    @pl.loop(0, BPW)
    def _zero(i):
        acc[i, pl.ds(0, LANES)] = zeros
        acc[i, pl.ds(LANES, LANES)] = zeros

    def accum(h, rows):
        for c in range(NGRP):
            chunk = ctx_v[h, pl.ds(c * LANES, LANES)]
            for l in range(LANES):
                slot = c * LANES + l
                off = lax.shift_left(lax.bitwise_and(chunk[l], 3), 5)
                ids = jnp.full((LANES,), slot, jnp.int32)
                offs = iota + off
                lo = plsc.load_gather(rows, [ids, offs])
                hi = plsc.load_gather(rows, [ids, offs + LANES])
                plsc.addupdate(acc.at[slot, pl.ds(0, LANES)], lo)
                plsc.addupdate(acc.at[slot, pl.ds(LANES, LANES)], hi)

    pltpu.async_copy(tbl4_hbm.at[big_v.at[0]], rows0, sem0)
    pltpu.async_copy(tbl4_hbm.at[big_v.at[1]], rows1, sem1)

    @pl.loop(0, HIST, step=2)
    def _h(h):
        for b, (rows, sem) in enumerate(((rows0, sem0), (rows1, sem1))):
            hc = h + b
            pltpu.make_async_copy(tbl4_hbm.at[big_v.at[hc]], rows, sem).wait()
            accum(hc, rows)

            @pl.when(hc + 2 < HIST)
            def _next():
                pltpu.async_copy(tbl4_hbm.at[big_v.at[hc + 2]], rows, sem)

    pltpu.sync_copy(acc, con_hbm.at[pl.ds(base, BPW)])


def _ctx_gather(ctxT, tbl4):
    f = pl.kernel(
        _ctx_body,
        out_type=jax.ShapeDtypeStruct((BATCH, HIDDEN), jnp.float32),
        mesh=_mesh(),
        compiler_params=pltpu.CompilerParams(needs_layout_passes=False),
        scratch_types=[
            pltpu.VMEM((HIST, BPW), jnp.int32),
            pltpu.VMEM((HIST, BPW), jnp.int32),
            pltpu.VMEM((BPW, 128), jnp.float32),
            pltpu.VMEM((BPW, 128), jnp.float32),
            pltpu.VMEM((BPW, HIDDEN), jnp.float32),
            pltpu.SemaphoreType.DMA,
            pltpu.SemaphoreType.DMA,
        ],
    )
    return f(ctxT, tbl4)


def _tgt_body(t_hbm, outT_hbm, tgt_hbm,
              tidx_v, blk0, blk1, tgt_v, sem0, sem1):
    base = _wid() * BPW
    pltpu.sync_copy(t_hbm.at[pl.ds(base, BPW)], tidx_v)
    iota = lax.iota(jnp.int32, LANES)

    def block_base(vs):
        vc = lax.min(vs, TAIL - 1)
        return pl.multiple_of(
            lax.shift_left(lax.shift_right_logical(vc, 7), 7), 128)

    def fetch(vs, blk, sem):
        pltpu.async_copy(outT_hbm.at[:, pl.ds(block_base(vs), 128)], blk, sem)

    def wait(blk, sem):
        pltpu.make_async_copy(outT_hbm.at[:, pl.ds(0, 128)], blk, sem).wait()

    def select(i, vs, blk):
        off = lax.min(vs, TAIL - 1) - block_base(vs)
        lo = plsc.load_gather(blk, [iota, jnp.full((LANES,), off, jnp.int32)])
        hi = plsc.load_gather(blk, [iota + LANES,
                                    jnp.full((LANES,), off, jnp.int32)])
        tgt_v[i, pl.ds(0, LANES)] = lo
        tgt_v[i, pl.ds(LANES, LANES)] = hi

    @pl.loop(0, NGRP)
    def _g(c):
        chunk = tidx_v[pl.ds(c * LANES, LANES)]
        fetch(chunk[0], blk0, sem0)
        for l in range(LANES):
            if l + 1 < LANES:
                fetch(chunk[l + 1], (blk0, blk1)[(l + 1) % 2],
                      (sem0, sem1)[(l + 1) % 2])
            blk, sem = (blk0, blk1)[l % 2], (sem0, sem1)[l % 2]
            wait(blk, sem)
            select(c * LANES + l, chunk[l], blk)

    pltpu.sync_copy(tgt_v, tgt_hbm.at[pl.ds(base, BPW)])


def _tgt_gather(t_flat, out_embT):
    f = pl.kernel(
        _tgt_body,
        out_type=jax.ShapeDtypeStruct((BATCH, HIDDEN), jnp.float32),
        mesh=_mesh(),
        compiler_params=pltpu.CompilerParams(needs_layout_passes=False),
        scratch_types=[
            pltpu.VMEM((BPW,), jnp.int32),
            pltpu.VMEM((HIDDEN, 128), jnp.float32),
            pltpu.VMEM((HIDDEN, 128), jnp.float32),
            pltpu.VMEM((BPW, HIDDEN), jnp.float32),
            pltpu.SemaphoreType.DMA,
            pltpu.SemaphoreType.DMA,
        ],
    )
    return f(t_flat, out_embT)


def _mm_body(a_ref, b_ref, o_ref):
    o_ref[...] = lax.dot_general(a_ref[...], b_ref[...],
                                 (((1,), (1,)), ((), ())),
                                 preferred_element_type=jnp.float32)


def _tc_matmul(con, tgt):
    blk = 1024
    return pl.pallas_call(
        _mm_body,
        grid=(BATCH // blk, BATCH // blk),
        in_specs=[pl.BlockSpec((blk, HIDDEN), lambda i, j: (i, 0)),
                  pl.BlockSpec((blk, HIDDEN), lambda i, j: (j, 0))],
        out_specs=pl.BlockSpec((blk, blk), lambda i, j: (i, j)),
        out_shape=jax.ShapeDtypeStruct((BATCH, BATCH), jnp.float32),
    )(con, tgt)


def kernel(contexts, t, in_emb, out_emb):
    ctxT = contexts.T
    t_flat = t.reshape(BATCH)
    in_embT = in_emb.T
    out_embT = out_emb.T
    tail_in = in_emb[TAIL:]
    tail_tab = out_emb[TAIL:]

    tbl4 = _repack(in_embT, tail_in)
    con = _ctx_gather(ctxT, tbl4)
    tgt = _tgt_gather(t_flat, out_embT)
    tail_rows = jnp.take(tail_tab,
                         jnp.clip(t_flat - TAIL, 0, VOCAB - TAIL - 1), axis=0)
    tgt = jnp.where((t_flat >= TAIL)[:, None], tail_rows, tgt)
    return _tc_matmul(con, tgt)


# speedup vs baseline: 3.6111x; 1.0031x over previous
import functools

import jax
import jax.numpy as jnp
from jax import lax
from jax.experimental import pallas as pl
from jax.experimental.pallas import tpu as pltpu
from jax.experimental.pallas import tpu_sc as plsc

VOCAB = 1_000_000
HIDDEN = 32
BATCH = 4096
HIST = 50
NC, NS, LANES = 2, 16, 16
NW = NC * NS
BPW = BATCH // NW
NGRP = BPW // LANES
TAIL = (VOCAB // 128) * 128
NBIG = VOCAB // 4
SLABW = 512
NSLAB = TAIL // SLABW
SPAD = SLABW + 1


def _wid():
    return lax.axis_index("s") * NC + lax.axis_index("c")


def _mesh():
    return plsc.VectorSubcoreMesh(core_axis_name="c", subcore_axis_name="s",
                                  num_cores=NC, num_subcores=NS)


def _t0_body(inT_hbm, tail_hbm, tbl4_hbm, sbuf0, sbuf1, obuf0, obuf1, tbuf,
             si0, si1, so0, so1):
    w = _wid()
    iota = lax.iota(jnp.int32, LANES)
    row_lo = iota
    row_hi = iota + LANES
    sbufs, obufs = (sbuf0, sbuf1), (obuf0, obuf1)
    sis, sos = (si0, si1), (so0, so1)
    npass = (NSLAB + NW - 1) // NW

    def start_in(p, par):
        @pl.when(w + p * NW < NSLAB)
        def _():
            pltpu.async_copy(inT_hbm.at[:, pl.ds((w + p * NW) * SLABW, SLABW)],
                             sbufs[par].at[:, pl.ds(0, SLABW)], sis[par])

    def wait_in(par):
        pltpu.make_async_copy(inT_hbm.at[:, pl.ds(0, SLABW)],
                              sbufs[par].at[:, pl.ds(0, SLABW)],
                              sis[par]).wait()

    def wait_out(par):
        pltpu.make_async_copy(obufs[par], tbl4_hbm.at[pl.ds(0, SLABW // 4)],
                              sos[par]).wait()

    def compute(par):
        sbuf, obuf = sbufs[par], obufs[par]

        @pl.loop(0, SLABW // 4, unroll=8)
        def _r(r):
            for q in range(8):
                k = q // 2
                rows = row_lo if q % 2 == 0 else row_hi
                col = jnp.full((LANES,), 4 * r + k, jnp.int32)
                obuf[r, pl.ds(q * LANES, LANES)] = \
                    plsc.load_gather(sbuf, [rows, col])

    start_in(0, 0)

    @pl.loop(0, (npass + 1) // 2)
    def _p2(p2):
        for par in range(2):
            p = p2 * 2 + par
            slab = w + p * NW

            @pl.when(slab < NSLAB)
            def _do():
                wait_in(par)
                start_in(p + 1, 1 - par)

                @pl.when(p >= 2)
                def _w():
                    wait_out(par)
                compute(par)
                pltpu.async_copy(obufs[par],
                                 tbl4_hbm.at[pl.ds(slab * (SLABW // 4),
                                                   SLABW // 4)], sos[par])

    for par in range(2):
        last = npass - 2 + par

        @pl.when(w + last * NW < NSLAB)
        def _fin():
            wait_out(par if last % 2 == par else 1 - par)

    @pl.when(w == NW - 1)
    def _tail():
        pltpu.sync_copy(tail_hbm, tbuf)
        for r in range(16):
            for q in range(8):
                k = q // 2
                h0 = 0 if q % 2 == 0 else LANES
                obuf0[r, pl.ds(q * LANES, LANES)] = \
                    tbuf[4 * r + k, pl.ds(h0, LANES)]
        pltpu.sync_copy(obuf0.at[pl.ds(0, 16)],
                        tbl4_hbm.at[pl.ds(NBIG - 16, 16)])


def _repack(in_embT, tail_in):
    f = pl.kernel(
        _t0_body,
        out_type=jax.ShapeDtypeStruct((NBIG, 128), jnp.float32),
        mesh=_mesh(),
        compiler_params=pltpu.CompilerParams(needs_layout_passes=False),
        scratch_types=[
            pltpu.VMEM((HIDDEN, SPAD), jnp.float32),
            pltpu.VMEM((HIDDEN, SPAD), jnp.float32),
            pltpu.VMEM((SLABW // 4, 128), jnp.float32),
            pltpu.VMEM((SLABW // 4, 128), jnp.float32),
            pltpu.VMEM((64, HIDDEN), jnp.float32),
            pltpu.SemaphoreType.DMA,
            pltpu.SemaphoreType.DMA,
            pltpu.SemaphoreType.DMA,
            pltpu.SemaphoreType.DMA,
        ],
    )
    return f(in_embT, tail_in)


def _ctx_body(ctxT_hbm, tbl4_hbm, con_hbm,
              ctx_v, big_v, rows0, rows1, acc, sem0, sem1):
    base = _wid() * BPW
    iota = lax.iota(jnp.int32, LANES)
    pltpu.sync_copy(ctxT_hbm.at[:, pl.ds(base, BPW)], ctx_v)

    @pl.loop(0, HIST)
    def _prep(h):
        for c in range(NGRP):
            sl = pl.ds(c * LANES, LANES)
            big_v[h, sl] = lax.shift_right_logical(ctx_v[h, sl], 2)

    zeros = jnp.zeros((LANES,), jnp.float32)

    @pl.loop(0, BPW)
    def _zero(i):
        acc[i, pl.ds(0, LANES)] = zeros
        acc[i, pl.ds(LANES, LANES)] = zeros

    def accum(h, rows):
        for c in range(NGRP):
            chunk = ctx_v[h, pl.ds(c * LANES, LANES)]
            for l in range(LANES):
                slot = c * LANES + l
                off = lax.shift_left(lax.bitwise_and(chunk[l], 3), 5)
                ids = jnp.full((LANES,), slot, jnp.int32)
                offs = iota + off
                lo = plsc.load_gather(rows, [ids, offs])
                hi = plsc.load_gather(rows, [ids, offs + LANES])
                plsc.addupdate(acc.at[slot, pl.ds(0, LANES)], lo)
                plsc.addupdate(acc.at[slot, pl.ds(LANES, LANES)], hi)

    pltpu.async_copy(tbl4_hbm.at[big_v.at[0]], rows0, sem0)
    pltpu.async_copy(tbl4_hbm.at[big_v.at[1]], rows1, sem1)

    @pl.loop(0, HIST, step=2)
    def _h(h):
        for b, (rows, sem) in enumerate(((rows0, sem0), (rows1, sem1))):
            hc = h + b
            pltpu.make_async_copy(tbl4_hbm.at[big_v.at[hc]], rows, sem).wait()
            accum(hc, rows)

            @pl.when(hc + 2 < HIST)
            def _next():
                pltpu.async_copy(tbl4_hbm.at[big_v.at[hc + 2]], rows, sem)

    pltpu.sync_copy(acc, con_hbm.at[pl.ds(base, BPW)])


def _ctx_gather(ctxT, tbl4):
    f = pl.kernel(
        _ctx_body,
        out_type=jax.ShapeDtypeStruct((BATCH, HIDDEN), jnp.float32),
        mesh=_mesh(),
        compiler_params=pltpu.CompilerParams(needs_layout_passes=False),
        scratch_types=[
            pltpu.VMEM((HIST, BPW), jnp.int32),
            pltpu.VMEM((HIST, BPW), jnp.int32),
            pltpu.VMEM((BPW, 128), jnp.float32),
            pltpu.VMEM((BPW, 128), jnp.float32),
            pltpu.VMEM((BPW, HIDDEN), jnp.float32),
            pltpu.SemaphoreType.DMA,
            pltpu.SemaphoreType.DMA,
        ],
    )
    return f(ctxT, tbl4)


def _tgt_body(t_hbm, outT_hbm, tgt_hbm,
              tidx_v, blk0, blk1, tgt_v, sem0, sem1):
    base = _wid() * BPW
    pltpu.sync_copy(t_hbm.at[pl.ds(base, BPW)], tidx_v)
    iota = lax.iota(jnp.int32, LANES)

    def block_base(vs):
        vc = lax.min(vs, TAIL - 1)
        return pl.multiple_of(
            lax.shift_left(lax.shift_right_logical(vc, 7), 7), 128)

    def fetch(vs, blk, sem):
        pltpu.async_copy(outT_hbm.at[:, pl.ds(block_base(vs), 128)], blk, sem)

    def wait(blk, sem):
        pltpu.make_async_copy(outT_hbm.at[:, pl.ds(0, 128)], blk, sem).wait()

    def select(i, vs, blk):
        off = lax.min(vs, TAIL - 1) - block_base(vs)
        lo = plsc.load_gather(blk, [iota, jnp.full((LANES,), off, jnp.int32)])
        hi = plsc.load_gather(blk, [iota + LANES,
                                    jnp.full((LANES,), off, jnp.int32)])
        tgt_v[i, pl.ds(0, LANES)] = lo
        tgt_v[i, pl.ds(LANES, LANES)] = hi

    @pl.loop(0, NGRP)
    def _g(c):
        chunk = tidx_v[pl.ds(c * LANES, LANES)]
        fetch(chunk[0], blk0, sem0)
        for l in range(LANES):
            if l + 1 < LANES:
                fetch(chunk[l + 1], (blk0, blk1)[(l + 1) % 2],
                      (sem0, sem1)[(l + 1) % 2])
            blk, sem = (blk0, blk1)[l % 2], (sem0, sem1)[l % 2]
            wait(blk, sem)
            select(c * LANES + l, chunk[l], blk)

    pltpu.sync_copy(tgt_v, tgt_hbm.at[pl.ds(base, BPW)])


def _tgt_gather(t_flat, out_embT):
    f = pl.kernel(
        _tgt_body,
        out_type=jax.ShapeDtypeStruct((BATCH, HIDDEN), jnp.float32),
        mesh=_mesh(),
        compiler_params=pltpu.CompilerParams(needs_layout_passes=False),
        scratch_types=[
            pltpu.VMEM((BPW,), jnp.int32),
            pltpu.VMEM((HIDDEN, 128), jnp.float32),
            pltpu.VMEM((HIDDEN, 128), jnp.float32),
            pltpu.VMEM((BPW, HIDDEN), jnp.float32),
            pltpu.SemaphoreType.DMA,
            pltpu.SemaphoreType.DMA,
        ],
    )
    return f(t_flat, out_embT)


def _mm_body(a_ref, b_ref, o_ref):
    o_ref[...] = lax.dot_general(a_ref[...], b_ref[...],
                                 (((1,), (1,)), ((), ())),
                                 preferred_element_type=jnp.float32)


def _tc_matmul(con, tgt):
    blk = 1024
    return pl.pallas_call(
        _mm_body,
        grid=(BATCH // blk, BATCH // blk),
        in_specs=[pl.BlockSpec((blk, HIDDEN), lambda i, j: (i, 0)),
                  pl.BlockSpec((blk, HIDDEN), lambda i, j: (j, 0))],
        out_specs=pl.BlockSpec((blk, blk), lambda i, j: (i, j)),
        out_shape=jax.ShapeDtypeStruct((BATCH, BATCH), jnp.float32),
    )(con, tgt)


def kernel(contexts, t, in_emb, out_emb):
    ctxT = contexts.T
    t_flat = t.reshape(BATCH)
    in_embT = in_emb.T
    out_embT = out_emb.T
    tail_in = in_emb[TAIL:]
    tail_tab = out_emb[TAIL:]

    tbl4 = _repack(in_embT, tail_in)
    con = _ctx_gather(ctxT, tbl4)
    tgt = _tgt_gather(t_flat, out_embT)
    tail_rows = jnp.take(tail_tab,
                         jnp.clip(t_flat - TAIL, 0, VOCAB - TAIL - 1), axis=0)
    tgt = jnp.where((t_flat >= TAIL)[:, None], tail_rows, tgt)
    return _tc_matmul(con, tgt)


# speedup vs baseline: 5.5506x; 1.5371x over previous
import functools

import jax
import jax.numpy as jnp
from jax import lax
from jax.experimental import pallas as pl
from jax.experimental.pallas import tpu as pltpu
from jax.experimental.pallas import tpu_sc as plsc

VOCAB = 1_000_000
HIDDEN = 32
BATCH = 4096
HIST = 50
NC, NS, LANES = 2, 16, 16
NW = NC * NS
BPW = BATCH // NW
NGRP = BPW // LANES
TAIL = (VOCAB // 128) * 128
NBIG = VOCAB // 4
SLABW = 512
NSLAB = TAIL // SLABW
SPAD = SLABW + 1


def _wid():
    return lax.axis_index("s") * NC + lax.axis_index("c")


def _mesh():
    return plsc.VectorSubcoreMesh(core_axis_name="c", subcore_axis_name="s",
                                  num_cores=NC, num_subcores=NS)


def _t0_body(inT_hbm, tail_hbm, tbl4_hbm, sbuf0, sbuf1, obuf0, obuf1, tbuf,
             si0, si1, so0, so1):
    w = _wid()
    iota = lax.iota(jnp.int32, LANES)
    row_lo = iota
    row_hi = iota + LANES
    sbufs, obufs = (sbuf0, sbuf1), (obuf0, obuf1)
    sis, sos = (si0, si1), (so0, so1)
    npass = (NSLAB + NW - 1) // NW

    def start_in(p, par):
        @pl.when(w + p * NW < NSLAB)
        def _():
            pltpu.async_copy(inT_hbm.at[:, pl.ds((w + p * NW) * SLABW, SLABW)],
                             sbufs[par].at[:, pl.ds(0, SLABW)], sis[par])

    def wait_in(par):
        pltpu.make_async_copy(inT_hbm.at[:, pl.ds(0, SLABW)],
                              sbufs[par].at[:, pl.ds(0, SLABW)],
                              sis[par]).wait()

    def wait_out(par):
        pltpu.make_async_copy(obufs[par], tbl4_hbm.at[pl.ds(0, SLABW // 4)],
                              sos[par]).wait()

    def compute(par):
        sbuf, obuf = sbufs[par], obufs[par]

        @pl.loop(0, SLABW // 4, unroll=8)
        def _r(r):
            for q in range(8):
                k = q // 2
                rows = row_lo if q % 2 == 0 else row_hi
                col = jnp.full((LANES,), 4 * r + k, jnp.int32)
                obuf[r, pl.ds(q * LANES, LANES)] = \
                    plsc.load_gather(sbuf, [rows, col])

    start_in(0, 0)

    @pl.loop(0, (npass + 1) // 2)
    def _p2(p2):
        for par in range(2):
            p = p2 * 2 + par
            slab = w + p * NW

            @pl.when(slab < NSLAB)
            def _do():
                wait_in(par)
                start_in(p + 1, 1 - par)

                @pl.when(p >= 2)
                def _w():
                    wait_out(par)
                compute(par)
                pltpu.async_copy(obufs[par],
                                 tbl4_hbm.at[pl.ds(slab * (SLABW // 4),
                                                   SLABW // 4)], sos[par])

    for par in range(2):
        last = npass - 2 + par

        @pl.when(w + last * NW < NSLAB)
        def _fin():
            wait_out(par if last % 2 == par else 1 - par)

    @pl.when(w == NW - 1)
    def _tail():
        pltpu.sync_copy(tail_hbm, tbuf)
        for r in range(16):
            for q in range(8):
                k = q // 2
                h0 = 0 if q % 2 == 0 else LANES
                obuf0[r, pl.ds(q * LANES, LANES)] = \
                    tbuf[4 * r + k, pl.ds(h0, LANES)]
        pltpu.sync_copy(obuf0.at[pl.ds(0, 16)],
                        tbl4_hbm.at[pl.ds(NBIG - 16, 16)])


def _repack(in_embT, tail_in):
    f = pl.kernel(
        _t0_body,
        out_type=jax.ShapeDtypeStruct((NBIG, 128), jnp.float32),
        mesh=_mesh(),
        compiler_params=pltpu.CompilerParams(needs_layout_passes=False),
        scratch_types=[
            pltpu.VMEM((HIDDEN, SPAD), jnp.float32),
            pltpu.VMEM((HIDDEN, SPAD), jnp.float32),
            pltpu.VMEM((SLABW // 4, 128), jnp.float32),
            pltpu.VMEM((SLABW // 4, 128), jnp.float32),
            pltpu.VMEM((64, HIDDEN), jnp.float32),
            pltpu.SemaphoreType.DMA,
            pltpu.SemaphoreType.DMA,
            pltpu.SemaphoreType.DMA,
            pltpu.SemaphoreType.DMA,
        ],
    )
    return f(in_embT, tail_in)


def _ctx_body(ctxT_hbm, tbl4_hbm, con_hbm,
              ctx_v, big_v, rows0, rows1, acc, sem0, sem1):
    base = _wid() * BPW
    iota = lax.iota(jnp.int32, LANES)
    pltpu.sync_copy(ctxT_hbm.at[:, pl.ds(base, BPW)], ctx_v)

    @pl.loop(0, HIST)
    def _prep(h):
        for c in range(NGRP):
            sl = pl.ds(c * LANES, LANES)
            big_v[h, sl] = lax.shift_right_logical(ctx_v[h, sl], 2)

    zeros = jnp.zeros((LANES,), jnp.float32)

    @pl.loop(0, BPW)
    def _zero(i):
        acc[i, pl.ds(0, LANES)] = zeros
        acc[i, pl.ds(LANES, LANES)] = zeros

    def accum(h, rows):
        for c in range(NGRP):
            chunk = ctx_v[h, pl.ds(c * LANES, LANES)]
            for l in range(LANES):
                slot = c * LANES + l
                off = lax.shift_left(lax.bitwise_and(chunk[l], 3), 5)
                ids = jnp.full((LANES,), slot, jnp.int32)
                offs = iota + off
                lo = plsc.load_gather(rows, [ids, offs])
                hi = plsc.load_gather(rows, [ids, offs + LANES])
                plsc.addupdate(acc.at[slot, pl.ds(0, LANES)], lo)
                plsc.addupdate(acc.at[slot, pl.ds(LANES, LANES)], hi)

    pltpu.async_copy(tbl4_hbm.at[big_v.at[0]], rows0, sem0)
    pltpu.async_copy(tbl4_hbm.at[big_v.at[1]], rows1, sem1)

    @pl.loop(0, HIST, step=2)
    def _h(h):
        for b, (rows, sem) in enumerate(((rows0, sem0), (rows1, sem1))):
            hc = h + b
            pltpu.make_async_copy(tbl4_hbm.at[big_v.at[hc]], rows, sem).wait()
            accum(hc, rows)

            @pl.when(hc + 2 < HIST)
            def _next():
                pltpu.async_copy(tbl4_hbm.at[big_v.at[hc + 2]], rows, sem)

    pltpu.sync_copy(acc, con_hbm.at[pl.ds(base, BPW)])


def _ctx_gather(ctxT, tbl4):
    f = pl.kernel(
        _ctx_body,
        out_type=jax.ShapeDtypeStruct((BATCH, HIDDEN), jnp.float32),
        mesh=_mesh(),
        compiler_params=pltpu.CompilerParams(needs_layout_passes=False),
        scratch_types=[
            pltpu.VMEM((HIST, BPW), jnp.int32),
            pltpu.VMEM((HIST, BPW), jnp.int32),
            pltpu.VMEM((BPW, 128), jnp.float32),
            pltpu.VMEM((BPW, 128), jnp.float32),
            pltpu.VMEM((BPW, HIDDEN), jnp.float32),
            pltpu.SemaphoreType.DMA,
            pltpu.SemaphoreType.DMA,
        ],
    )
    return f(ctxT, tbl4)


def _tgt_body(t_hbm, outT_hbm, tgt_hbm,
              tidx_v, blk0, blk1, tgt_v, sem0, sem1):
    base = _wid() * BPW
    pltpu.sync_copy(t_hbm.at[pl.ds(base, BPW)], tidx_v)
    iota = lax.iota(jnp.int32, LANES)

    def block_base(vs):
        vc = lax.min(vs, TAIL - 1)
        return pl.multiple_of(
            lax.shift_left(lax.shift_right_logical(vc, 7), 7), 128)

    def fetch(vs, blk, sem):
        pltpu.async_copy(outT_hbm.at[:, pl.ds(block_base(vs), 128)], blk, sem)

    def wait(blk, sem):
        pltpu.make_async_copy(outT_hbm.at[:, pl.ds(0, 128)], blk, sem).wait()

    def select(i, vs, blk):
        off = lax.min(vs, TAIL - 1) - block_base(vs)
        lo = plsc.load_gather(blk, [iota, jnp.full((LANES,), off, jnp.int32)])
        hi = plsc.load_gather(blk, [iota + LANES,
                                    jnp.full((LANES,), off, jnp.int32)])
        tgt_v[i, pl.ds(0, LANES)] = lo
        tgt_v[i, pl.ds(LANES, LANES)] = hi

    @pl.loop(0, NGRP)
    def _g(c):
        chunk = tidx_v[pl.ds(c * LANES, LANES)]
        fetch(chunk[0], blk0, sem0)
        for l in range(LANES):
            if l + 1 < LANES:
                fetch(chunk[l + 1], (blk0, blk1)[(l + 1) % 2],
                      (sem0, sem1)[(l + 1) % 2])
            blk, sem = (blk0, blk1)[l % 2], (sem0, sem1)[l % 2]
            wait(blk, sem)
            select(c * LANES + l, chunk[l], blk)

    pltpu.sync_copy(tgt_v, tgt_hbm.at[pl.ds(base, BPW)])


def _tgt_gather(t_flat, out_embT):
    f = pl.kernel(
        _tgt_body,
        out_type=jax.ShapeDtypeStruct((BATCH, HIDDEN), jnp.float32),
        mesh=_mesh(),
        compiler_params=pltpu.CompilerParams(needs_layout_passes=False),
        scratch_types=[
            pltpu.VMEM((BPW,), jnp.int32),
            pltpu.VMEM((HIDDEN, 128), jnp.float32),
            pltpu.VMEM((HIDDEN, 128), jnp.float32),
            pltpu.VMEM((BPW, HIDDEN), jnp.float32),
            pltpu.SemaphoreType.DMA,
            pltpu.SemaphoreType.DMA,
        ],
    )
    return f(t_flat, out_embT)


def _mm_body(a_ref, b_ref, o_ref):
    o_ref[...] = lax.dot_general(a_ref[...], b_ref[...],
                                 (((1,), (1,)), ((), ())),
                                 preferred_element_type=jnp.float32)


def _tc_matmul(con, tgt):
    blk = 1024
    return pl.pallas_call(
        _mm_body,
        grid=(BATCH // blk, BATCH // blk),
        in_specs=[pl.BlockSpec((blk, HIDDEN), lambda i, j: (i, 0)),
                  pl.BlockSpec((blk, HIDDEN), lambda i, j: (j, 0))],
        out_specs=pl.BlockSpec((blk, blk), lambda i, j: (i, j)),
        out_shape=jax.ShapeDtypeStruct((BATCH, BATCH), jnp.float32),
    )(con, tgt)


def kernel(contexts, t, in_emb, out_emb):
    ctxT = contexts.T
    t_flat = t.reshape(BATCH)
    out_embT = out_emb.T
    tail_tab = out_emb[TAIL:]

    tbl4 = in_emb.reshape(NBIG, 128)
    con = _ctx_gather(ctxT, tbl4)
    tgt = _tgt_gather(t_flat, out_embT)
    tail_rows = jnp.take(tail_tab,
                         jnp.clip(t_flat - TAIL, 0, VOCAB - TAIL - 1), axis=0)
    tgt = jnp.where((t_flat >= TAIL)[:, None], tail_rows, tgt)
    return _tc_matmul(con, tgt)
